# baseline (speedup 1.0000x reference)
import functools

import jax
import jax.numpy as jnp
from jax import lax
from jax.experimental import pallas as pl
from jax.experimental.pallas import tpu as pltpu
from jax.experimental.pallas import tpu_sc as plsc

NC = 2
NS = 16
LANES = 16
CHUNK = 128
CG = 64

_SC_PARAMS = pltpu.CompilerParams(use_tc_tiling_on_sc=False)


def _pad_to(v, m):
    return (v + m - 1) // m * m


def _fill_iota(iota_v, base):
    lanes_iota = lax.iota(jnp.int32, LANES)
    for t in range(CHUNK // LANES):
        iota_v[pl.ds(t * LANES, LANES)] = lanes_iota + (base + t * LANES)


def _deg_kernel_body(npad, ea_per_tile, dst_hbm, degp_hbm,
                     idx_v, ones_v, zbuf_v, iota_v, obuf_v, deg_sh):
    c = lax.axis_index("c")
    s = lax.axis_index("s")
    rows_per_tile = npad // NS

    def fill_ones(i, _):
        ones_v[i] = jnp.ones((LANES,), jnp.float32)
        return _
    lax.fori_loop(0, CHUNK, fill_ones, None)

    def fill_zeros(i, _):
        zbuf_v[i] = jnp.zeros((LANES,), jnp.float32)
        return _
    lax.fori_loop(0, CHUNK, fill_zeros, None)

    for k in range(rows_per_tile // CHUNK):
        _fill_iota(iota_v, s * rows_per_tile + k * CHUNK)
        pltpu.sync_copy(zbuf_v, deg_sh.at[iota_v])

    plsc.subcore_barrier()

    base = (c * NS + s) * ea_per_tile

    def chunk_body(j, _):
        pltpu.sync_copy(dst_hbm.at[pl.ds(base + j * CHUNK, CHUNK)], idx_v)
        pltpu.sync_copy(ones_v, deg_sh.at[idx_v], add=True)
        return _
    lax.fori_loop(0, ea_per_tile // CHUNK, chunk_body, None)

    plsc.subcore_barrier()

    for k in range(rows_per_tile // CHUNK):
        row0 = s * rows_per_tile + k * CHUNK
        _fill_iota(iota_v, row0)
        pltpu.sync_copy(deg_sh.at[iota_v], obuf_v)
        pltpu.sync_copy(obuf_v, degp_hbm.at[pl.ds(c * npad + row0, CHUNK)])


def _make_deg_kernel(npad, epad):
    ea_per_tile = epad // (NC * NS)
    mesh = plsc.VectorSubcoreMesh(core_axis_name="c", subcore_axis_name="s")
    return pl.kernel(
        functools.partial(_deg_kernel_body, npad, ea_per_tile),
        out_type=jax.ShapeDtypeStruct((NC * npad, LANES), jnp.float32),
        mesh=mesh,
        scratch_types=[
            pltpu.VMEM((CHUNK,), jnp.int32),
            pltpu.VMEM((CHUNK, LANES), jnp.float32),
            pltpu.VMEM((CHUNK, LANES), jnp.float32),
            pltpu.VMEM((CHUNK,), jnp.int32),
            pltpu.VMEM((CHUNK, LANES), jnp.float32),
            pltpu.VMEM_SHARED((npad, LANES), jnp.float32),
        ],
        compiler_params=_SC_PARAMS,
    )


def _scatter_kernel_body(npad, ec_per_tile,
                         y0_hbm, y1_hbm, srcp_hbm, dstp_hbm,
                         acc0_hbm, acc1_hbm,
                         src2_v, dst2_v, rows_a, rows_b, iota_v,
                         sem_a, sem_b, acc_sh):
    c = lax.axis_index("c")
    s = lax.axis_index("s")
    rows_per_tile = npad // NS
    nchunks = ec_per_tile // CG

    def fill_iota64(base):
        lanes_iota = lax.iota(jnp.int32, LANES)
        for t in range(CG // LANES):
            iota_v[pl.ds(t * LANES, LANES)] = lanes_iota + (base + t * LANES)

    def fill_zeros(i, _):
        for jj in range(128 // LANES):
            rows_a[i, pl.ds(jj * LANES, LANES)] = jnp.zeros(
                (LANES,), jnp.float32)
        return _
    lax.fori_loop(0, CG, fill_zeros, None)

    pltpu.sync_copy(srcp_hbm.at[s], src2_v)
    pltpu.sync_copy(dstp_hbm.at[s], dst2_v)

    for k in range(rows_per_tile // CG):
        fill_iota64(s * rows_per_tile + k * CG)
        pltpu.sync_copy(rows_a, acc_sh.at[iota_v])
    plsc.subcore_barrier()

    def start_g(j, buf, sem):
        @pl.when(c == 0)
        def _g0():
            pltpu.async_copy(y0_hbm.at[src2_v.at[j]], buf, sem)

        @pl.when(c == 1)
        def _g1():
            pltpu.async_copy(y1_hbm.at[src2_v.at[j]], buf, sem)

    def wait_g(buf, sem):
        pltpu.make_async_copy(y0_hbm.at[src2_v.at[0]], buf, sem).wait()

    start_g(0, rows_a, sem_a)

    def pair_body(j, _):
        e0 = 2 * j
        wait_g(rows_a, sem_a)
        start_g(e0 + 1, rows_b, sem_b)
        pltpu.sync_copy(rows_a, acc_sh.at[dst2_v.at[e0]], add=True)
        wait_g(rows_b, sem_b)
        start_g(e0 + 2, rows_a, sem_a)
        pltpu.sync_copy(rows_b, acc_sh.at[dst2_v.at[e0 + 1]], add=True)
        return _
    lax.fori_loop(0, nchunks // 2, pair_body, None)
    wait_g(rows_a, sem_a)

    plsc.subcore_barrier()

    for k in range(rows_per_tile // CG):
        row0 = s * rows_per_tile + k * CG
        fill_iota64(row0)
        pltpu.sync_copy(acc_sh.at[iota_v], rows_a)

        @pl.when(c == 0)
        def _o0():
            pltpu.sync_copy(rows_a, acc0_hbm.at[pl.ds(row0, CG)])

        @pl.when(c == 1)
        def _o1():
            pltpu.sync_copy(rows_a, acc1_hbm.at[pl.ds(row0, CG)])


def _make_scatter_kernel(npad, epad):
    ec_per_tile = epad // NS
    nchunks = ec_per_tile // CG
    mesh = plsc.VectorSubcoreMesh(core_axis_name="c", subcore_axis_name="s")
    return pl.kernel(
        functools.partial(_scatter_kernel_body, npad, ec_per_tile),
        out_type=(
            jax.ShapeDtypeStruct((npad, 128), jnp.float32),
            jax.ShapeDtypeStruct((npad, 128), jnp.float32),
        ),
        mesh=mesh,
        scratch_types=[
            pltpu.VMEM((nchunks + 1, CG), jnp.int32),
            pltpu.VMEM((nchunks, CG), jnp.int32),
            pltpu.VMEM((CG, 128), jnp.float32),
            pltpu.VMEM((CG, 128), jnp.float32),
            pltpu.VMEM((CG,), jnp.int32),
            pltpu.SemaphoreType.DMA,
            pltpu.SemaphoreType.DMA,
            pltpu.VMEM_SHARED((npad, 128), jnp.float32),
        ],
        compiler_params=_SC_PARAMS,
    )


def _linear_body(x_ref, w_ref, degp_ref, y0_ref, y1_ref):
    deg = degp_ref[0, :, 0:1] + degp_ref[1, :, 0:1] + 1.0
    dis = lax.rsqrt(deg)
    xw = jnp.dot(x_ref[...], w_ref[...], preferred_element_type=jnp.float32)
    y0_ref[...] = xw[:, :128] * dis
    y1_ref[...] = xw[:, 128:] * dis


def _epilogue_body(acc0_ref, acc1_ref, y0_ref, y1_ref, degp_ref, b_ref,
                   out_ref):
    deg = degp_ref[0, :, 0:1] + degp_ref[1, :, 0:1] + 1.0
    dis = lax.rsqrt(deg)
    left = (acc0_ref[...] + y0_ref[...]) * dis
    right = (acc1_ref[...] + y1_ref[...]) * dis
    out_ref[...] = jnp.concatenate([left, right], axis=1) + b_ref[...][None, :]


def kernel(x, edge_index, W, b):
    n, f = x.shape
    h = W.shape[1]
    e = edge_index.shape[1]
    assert f == 256 and h == 256

    npad = _pad_to(n + 1, NS * CHUNK)
    epad = _pad_to(e, NC * NS * CHUNK)

    src = edge_index[0]
    dst = edge_index[1]
    pad_e = epad - e
    srcp = jnp.concatenate([src, jnp.zeros((pad_e,), jnp.int32)])
    dstp = jnp.concatenate([dst, jnp.full((pad_e,), n, jnp.int32)])

    degp = _make_deg_kernel(npad, epad)(dstp).reshape(NC, npad, LANES)

    rb = 400
    assert n % rb == 0
    grid = n // rb
    y0, y1 = pl.pallas_call(
        _linear_body,
        grid=(grid,),
        in_specs=[
            pl.BlockSpec((rb, f), lambda i: (i, 0)),
            pl.BlockSpec((f, h), lambda i: (0, 0)),
            pl.BlockSpec((NC, rb, LANES), lambda i: (0, i, 0)),
        ],
        out_specs=[
            pl.BlockSpec((rb, 128), lambda i: (i, 0)),
            pl.BlockSpec((rb, 128), lambda i: (i, 0)),
        ],
        out_shape=[
            jax.ShapeDtypeStruct((n, 128), jnp.float32),
            jax.ShapeDtypeStruct((n, 128), jnp.float32),
        ],
    )(x, W, degp)

    nchunks = epad // NS // CG
    src3 = jnp.concatenate(
        [srcp.reshape(NS, nchunks, CG),
         jnp.zeros((NS, 1, CG), jnp.int32)], axis=1)
    dst3 = dstp.reshape(NS, nchunks, CG)
    acc0, acc1 = _make_scatter_kernel(npad, epad)(y0, y1, src3, dst3)

    out = pl.pallas_call(
        _epilogue_body,
        grid=(grid,),
        in_specs=[
            pl.BlockSpec((rb, 128), lambda i: (i, 0)),
            pl.BlockSpec((rb, 128), lambda i: (i, 0)),
            pl.BlockSpec((rb, 128), lambda i: (i, 0)),
            pl.BlockSpec((rb, 128), lambda i: (i, 0)),
            pl.BlockSpec((NC, rb, LANES), lambda i: (0, i, 0)),
            pl.BlockSpec((h,), lambda i: (0,)),
        ],
        out_specs=pl.BlockSpec((rb, h), lambda i: (i, 0)),
        out_shape=jax.ShapeDtypeStruct((n, h), jnp.float32),
    )(acc0, acc1, y0, y1, degp, b)
    return out


# speedup vs baseline: 1.0300x; 1.0300x over previous
import functools

import jax
import jax.numpy as jnp
from jax import lax
from jax.experimental import pallas as pl
from jax.experimental.pallas import tpu as pltpu
from jax.experimental.pallas import tpu_sc as plsc

NC = 2
NS = 16
LANES = 16
CHUNK = 128

_SC_PARAMS = pltpu.CompilerParams(use_tc_tiling_on_sc=False)


def _pad_to(v, m):
    return (v + m - 1) // m * m


def _fill_iota(iota_v, base):
    lanes_iota = lax.iota(jnp.int32, LANES)
    for t in range(CHUNK // LANES):
        iota_v[pl.ds(t * LANES, LANES)] = lanes_iota + (base + t * LANES)


def _deg_kernel_body(npad, ea_per_tile, dst_hbm, degp_hbm,
                     idx_v, ones_v, zbuf_v, iota_v, obuf_v, deg_sh):
    c = lax.axis_index("c")
    s = lax.axis_index("s")
    rows_per_tile = npad // NS

    def fill_ones(i, _):
        ones_v[i] = jnp.ones((LANES,), jnp.float32)
        return _
    lax.fori_loop(0, CHUNK, fill_ones, None)

    def fill_zeros(i, _):
        zbuf_v[i] = jnp.zeros((LANES,), jnp.float32)
        return _
    lax.fori_loop(0, CHUNK, fill_zeros, None)

    for k in range(rows_per_tile // CHUNK):
        _fill_iota(iota_v, s * rows_per_tile + k * CHUNK)
        pltpu.sync_copy(zbuf_v, deg_sh.at[iota_v])

    plsc.subcore_barrier()

    base = (c * NS + s) * ea_per_tile

    def chunk_body(j, _):
        pltpu.sync_copy(dst_hbm.at[pl.ds(base + j * CHUNK, CHUNK)], idx_v)
        pltpu.sync_copy(ones_v, deg_sh.at[idx_v], add=True)
        return _
    lax.fori_loop(0, ea_per_tile // CHUNK, chunk_body, None)

    plsc.subcore_barrier()

    for k in range(rows_per_tile // CHUNK):
        row0 = s * rows_per_tile + k * CHUNK
        _fill_iota(iota_v, row0)
        pltpu.sync_copy(deg_sh.at[iota_v], obuf_v)
        pltpu.sync_copy(obuf_v, degp_hbm.at[pl.ds(c * npad + row0, CHUNK)])


def _make_deg_kernel(npad, epad):
    ea_per_tile = epad // (NC * NS)
    mesh = plsc.VectorSubcoreMesh(core_axis_name="c", subcore_axis_name="s")
    return pl.kernel(
        functools.partial(_deg_kernel_body, npad, ea_per_tile),
        out_type=jax.ShapeDtypeStruct((NC * npad, LANES), jnp.float32),
        mesh=mesh,
        scratch_types=[
            pltpu.VMEM((CHUNK,), jnp.int32),
            pltpu.VMEM((CHUNK, LANES), jnp.float32),
            pltpu.VMEM((CHUNK, LANES), jnp.float32),
            pltpu.VMEM((CHUNK,), jnp.int32),
            pltpu.VMEM((CHUNK, LANES), jnp.float32),
            pltpu.VMEM_SHARED((npad, LANES), jnp.float32),
        ],
        compiler_params=_SC_PARAMS,
    )


def _scatter_kernel_body(npad, ec_per_tile,
                         y0_hbm, y1_hbm, srcp_hbm, dstp_hbm,
                         acc0_hbm, acc1_hbm,
                         src2_v, dst2_v, rows_v, iota_v, acc_sh):
    c = lax.axis_index("c")
    s = lax.axis_index("s")
    rows_per_tile = npad // NS
    nchunks = ec_per_tile // CHUNK

    def fill_zeros(i, _):
        for jj in range(128 // LANES):
            rows_v[i, pl.ds(jj * LANES, LANES)] = jnp.zeros(
                (LANES,), jnp.float32)
        return _
    lax.fori_loop(0, CHUNK, fill_zeros, None)

    pltpu.sync_copy(srcp_hbm.at[s], src2_v)
    pltpu.sync_copy(dstp_hbm.at[s], dst2_v)

    for k in range(rows_per_tile // CHUNK):
        _fill_iota(iota_v, s * rows_per_tile + k * CHUNK)
        pltpu.sync_copy(rows_v, acc_sh.at[iota_v])
    plsc.subcore_barrier()

    def chunk_body(j, _):
        @pl.when(c == 0)
        def _g0():
            pltpu.sync_copy(y0_hbm.at[src2_v.at[j]], rows_v)

        @pl.when(c == 1)
        def _g1():
            pltpu.sync_copy(y1_hbm.at[src2_v.at[j]], rows_v)

        pltpu.sync_copy(rows_v, acc_sh.at[dst2_v.at[j]], add=True)
        return _
    lax.fori_loop(0, nchunks, chunk_body, None)

    plsc.subcore_barrier()

    for k in range(rows_per_tile // CHUNK):
        row0 = s * rows_per_tile + k * CHUNK
        _fill_iota(iota_v, row0)
        pltpu.sync_copy(acc_sh.at[iota_v], rows_v)

        @pl.when(c == 0)
        def _o0():
            pltpu.sync_copy(rows_v, acc0_hbm.at[pl.ds(row0, CHUNK)])

        @pl.when(c == 1)
        def _o1():
            pltpu.sync_copy(rows_v, acc1_hbm.at[pl.ds(row0, CHUNK)])


def _make_scatter_kernel(npad, epad):
    ec_per_tile = epad // NS
    nchunks = ec_per_tile // CHUNK
    mesh = plsc.VectorSubcoreMesh(core_axis_name="c", subcore_axis_name="s")
    return pl.kernel(
        functools.partial(_scatter_kernel_body, npad, ec_per_tile),
        out_type=(
            jax.ShapeDtypeStruct((npad, 128), jnp.float32),
            jax.ShapeDtypeStruct((npad, 128), jnp.float32),
        ),
        mesh=mesh,
        scratch_types=[
            pltpu.VMEM((nchunks, CHUNK), jnp.int32),
            pltpu.VMEM((nchunks, CHUNK), jnp.int32),
            pltpu.VMEM((CHUNK, 128), jnp.float32),
            pltpu.VMEM((CHUNK,), jnp.int32),
            pltpu.VMEM_SHARED((npad, 128), jnp.float32),
        ],
        compiler_params=_SC_PARAMS,
    )


def _linear_body(x_ref, w_ref, degp_ref, y0_ref, y1_ref):
    deg = degp_ref[0, :, 0:1] + degp_ref[1, :, 0:1] + 1.0
    dis = lax.rsqrt(deg)
    xw = jnp.dot(x_ref[...], w_ref[...], preferred_element_type=jnp.float32)
    y0_ref[...] = xw[:, :128] * dis
    y1_ref[...] = xw[:, 128:] * dis


def _epilogue_body(acc0_ref, acc1_ref, y0_ref, y1_ref, degp_ref, b_ref,
                   out_ref):
    deg = degp_ref[0, :, 0:1] + degp_ref[1, :, 0:1] + 1.0
    dis = lax.rsqrt(deg)
    left = (acc0_ref[...] + y0_ref[...]) * dis
    right = (acc1_ref[...] + y1_ref[...]) * dis
    out_ref[...] = jnp.concatenate([left, right], axis=1) + b_ref[...][None, :]


def kernel(x, edge_index, W, b):
    n, f = x.shape
    h = W.shape[1]
    e = edge_index.shape[1]
    assert f == 256 and h == 256

    npad = _pad_to(n + 1, NS * CHUNK)
    epad = _pad_to(e, NC * NS * CHUNK)

    src = edge_index[0]
    dst = edge_index[1]
    pad_e = epad - e
    srcp = jnp.concatenate([src, jnp.zeros((pad_e,), jnp.int32)])
    dstp = jnp.concatenate([dst, jnp.full((pad_e,), n, jnp.int32)])

    degp = _make_deg_kernel(npad, epad)(dstp).reshape(NC, npad, LANES)

    rb = 400
    assert n % rb == 0
    grid = n // rb
    y0, y1 = pl.pallas_call(
        _linear_body,
        grid=(grid,),
        in_specs=[
            pl.BlockSpec((rb, f), lambda i: (i, 0)),
            pl.BlockSpec((f, h), lambda i: (0, 0)),
            pl.BlockSpec((NC, rb, LANES), lambda i: (0, i, 0)),
        ],
        out_specs=[
            pl.BlockSpec((rb, 128), lambda i: (i, 0)),
            pl.BlockSpec((rb, 128), lambda i: (i, 0)),
        ],
        out_shape=[
            jax.ShapeDtypeStruct((n, 128), jnp.float32),
            jax.ShapeDtypeStruct((n, 128), jnp.float32),
        ],
    )(x, W, degp)

    src3 = srcp.reshape(NS, epad // NS // CHUNK, CHUNK)
    dst3 = dstp.reshape(NS, epad // NS // CHUNK, CHUNK)
    acc0, acc1 = _make_scatter_kernel(npad, epad)(y0, y1, src3, dst3)

    out = pl.pallas_call(
        _epilogue_body,
        grid=(grid,),
        in_specs=[
            pl.BlockSpec((rb, 128), lambda i: (i, 0)),
            pl.BlockSpec((rb, 128), lambda i: (i, 0)),
            pl.BlockSpec((rb, 128), lambda i: (i, 0)),
            pl.BlockSpec((rb, 128), lambda i: (i, 0)),
            pl.BlockSpec((NC, rb, LANES), lambda i: (0, i, 0)),
            pl.BlockSpec((h,), lambda i: (0,)),
        ],
        out_specs=pl.BlockSpec((rb, h), lambda i: (i, 0)),
        out_shape=jax.ShapeDtypeStruct((n, h), jnp.float32),
    )(acc0, acc1, y0, y1, degp, b)
    return out
